# baseline (speedup 1.0000x reference)
import dataclasses
import functools

import jax
import jax.numpy as jnp
from jax.experimental import pallas as pl
from jax.experimental.pallas import tpu as pltpu
from jax.experimental.pallas import tpu_sc as plsc

N = 1048576
DIM = 64
BINS = (1, 2, 3, 4, 8, 16, 32, 64)
LANES = 16
TI = N // 128


def _bucket(v):
    acc = (v >= BINS[0]).astype(jnp.int32)
    for b in BINS[1:]:
        acc += (v >= b).astype(jnp.int32)
    return acc


def kernel(lengths, table):
    lengths = lengths.astype(jnp.int32).reshape(1, N)
    tab_t = table.T.reshape(DIM, 9)

    mesh = plsc.VectorSubcoreMesh(core_axis_name="c", subcore_axis_name="s")
    cp = pltpu.CompilerParams()
    if "needs_layout_passes" in pltpu.CompilerParams.__dataclass_fields__:
        cp = dataclasses.replace(cp, needs_layout_passes=False)
    cp = dataclasses.replace(cp, use_tc_tiling_on_sc=False)

    @functools.partial(
        pl.kernel,
        out_type=jax.ShapeDtypeStruct((8, TI, 8, 128), jnp.float32),
        mesh=mesh,
        scratch_types=[pltpu.VMEM((DIM, 9), jnp.float32)],
        compiler_params=cp,
    )
    def k(len_hbm, tab_hbm, out_hbm, tab_v):
        pltpu.sync_copy(tab_hbm, tab_v)

        def body(len_vmem, out_vmem):
            for j in range(2):
                @plsc.parallel_loop(0, 128, LANES, unroll=3)
                def _(c, j=j):
                    r = _bucket(len_vmem[0, pl.ds(j * 128 + c, LANES)])
                    for td in range(8):
                        for di in range(8):
                            out_vmem[td, j, di, pl.ds(c, LANES)] = (
                                plsc.load_gather(tab_v.at[td * 8 + di], [r])
                            )

        pltpu.emit_pipeline(
            body,
            grid=(TI // 2,),
            in_specs=[pl.BlockSpec((1, 256), lambda i: (0, i))],
            out_specs=[pl.BlockSpec((8, 2, 8, 128), lambda i: (0, i, 0, 0))],
            core_axis_name=("c", "s"),
            dimension_semantics=(pltpu.PARALLEL,),
        )(len_hbm, out_hbm)

    out4 = k(lengths, tab_t)
    return out4.transpose(1, 3, 0, 2).reshape(N, DIM)


# speedup vs baseline: 2.4003x; 2.4003x over previous
import dataclasses
import functools

import jax
import jax.numpy as jnp
from jax.experimental import pallas as pl
from jax.experimental.pallas import tpu as pltpu
from jax.experimental.pallas import tpu_sc as plsc

N = 1048576
DIM = 64
BINS = (1, 2, 3, 4, 8, 16, 32, 64)
LANES = 16
TI = N // 128


def _bucket(v):
    acc = (v >= BINS[0]).astype(jnp.int32)
    for b in BINS[1:]:
        acc += (v >= b).astype(jnp.int32)
    return acc


def kernel(lengths, table):
    lengths = lengths.astype(jnp.int32).reshape(1, N)
    tab_t = table.T.reshape(DIM, 9)

    mesh = plsc.VectorSubcoreMesh(core_axis_name="c", subcore_axis_name="s")
    cp = pltpu.CompilerParams()
    if "needs_layout_passes" in pltpu.CompilerParams.__dataclass_fields__:
        cp = dataclasses.replace(cp, needs_layout_passes=False)
    cp = dataclasses.replace(cp, use_tc_tiling_on_sc=False)

    @functools.partial(
        pl.kernel,
        out_type=jax.ShapeDtypeStruct((8, TI, 8, 128), jnp.float32),
        mesh=mesh,
        scratch_types=[pltpu.VMEM((DIM, 9), jnp.float32)],
        compiler_params=cp,
    )
    def k(len_hbm, tab_hbm, out_hbm, tab_v):
        pltpu.sync_copy(tab_hbm, tab_v)

        def body(len_vmem, out_vmem):
            for j in range(2):
                @plsc.parallel_loop(0, 128, LANES, unroll=2)
                def _(c, j=j):
                    r = _bucket(len_vmem[0, pl.ds(j * 128 + c, LANES)])
                    for td in range(8):
                        for di in range(8):
                            out_vmem[td, j, di, pl.ds(c, LANES)] = (
                                plsc.load_gather(tab_v.at[td * 8 + di], [r])
                            )

        pltpu.emit_pipeline(
            body,
            grid=(TI // 2,),
            in_specs=[pl.BlockSpec((1, 256), lambda i: (0, i))],
            out_specs=[pl.BlockSpec((8, 2, 8, 128), lambda i: (0, i, 0, 0))],
            core_axis_name=("c", "s"),
            dimension_semantics=(pltpu.PARALLEL,),
        )(len_hbm, out_hbm)

    out4 = k(lengths, tab_t)
    return out4.transpose(1, 3, 0, 2).reshape(N, DIM)
